# baseline (speedup 1.0000x reference)
import functools

import jax
import jax.numpy as jnp
from jax import lax
from jax.experimental import pallas as pl
from jax.experimental.pallas import tpu as pltpu
from jax.experimental.pallas import tpu_sc as plsc

NUM_SAMPLES = 100000
NUM_FEATURES = 1024
BATCH = 128
TEMP = 0.05
BLK = 2000

_N_STEPS = NUM_SAMPLES // BLK
_ROWS_PER_W = 16


def _sc_gather_body(tgt_hbm, f_rgb_hbm, f_nir_hbm, f_tir_hbm,
                    g_rgb_hbm, g_nir_hbm, g_tir_hbm,
                    idx_v, rows_v, sem):
    num_cores = 2
    wid = lax.axis_index("s") * num_cores + lax.axis_index("c")
    bank = wid // 8
    base = (wid % 8) * _ROWS_PER_W

    @pl.when(wid < 24)
    def _():
        pltpu.sync_copy(tgt_hbm.at[pl.ds(base, _ROWS_PER_W)], idx_v)

        @pl.when(bank == 0)
        def _():
            pltpu.async_copy(f_rgb_hbm.at[idx_v], rows_v, sem).wait()
            pltpu.sync_copy(rows_v, g_rgb_hbm.at[pl.ds(base, _ROWS_PER_W)])

        @pl.when(bank == 1)
        def _():
            pltpu.async_copy(f_nir_hbm.at[idx_v], rows_v, sem).wait()
            pltpu.sync_copy(rows_v, g_nir_hbm.at[pl.ds(base, _ROWS_PER_W)])

        @pl.when(bank == 2)
        def _():
            pltpu.async_copy(f_tir_hbm.at[idx_v], rows_v, sem).wait()
            pltpu.sync_copy(rows_v, g_tir_hbm.at[pl.ds(base, _ROWS_PER_W)])


def _sc_gather(targets, features_rgb, features_nir, features_tir):
    mesh = plsc.VectorSubcoreMesh(core_axis_name="c", subcore_axis_name="s")
    row = jax.ShapeDtypeStruct((BATCH, NUM_FEATURES), jnp.float32)
    fn = pl.kernel(
        _sc_gather_body,
        mesh=mesh,
        out_type=(row, row, row),
        scratch_types=[
            pltpu.VMEM((_ROWS_PER_W,), jnp.int32),
            pltpu.VMEM((_ROWS_PER_W, NUM_FEATURES), jnp.float32),
            pltpu.SemaphoreType.DMA,
        ],
    )
    return fn(targets, features_rgb, features_nir, features_tir)


def _lse_kernel(x_rgb_ref, x_nir_ref, x_tir_ref,
                f_rgb_ref, f_nir_ref, f_tir_ref,
                lse_ref, xn_rgb, xn_nir, xn_tir, se):
    j = pl.program_id(0)

    @pl.when(j == 0)
    def _init():
        for src, dst in ((x_rgb_ref, xn_rgb), (x_nir_ref, xn_nir),
                         (x_tir_ref, xn_tir)):
            x = src[...]
            n = jnp.sqrt(jnp.sum(x * x, axis=1, keepdims=True))
            dst[...] = x / jnp.maximum(n, 1e-12)
        se[...] = jnp.zeros_like(se)

    inv_t = 1.0 / TEMP
    for k, (xn, f_ref) in enumerate(((xn_rgb, f_rgb_ref), (xn_nir, f_nir_ref),
                                     (xn_tir, f_tir_ref))):
        d = jax.lax.dot_general(
            xn[...].astype(jnp.bfloat16), f_ref[...].astype(jnp.bfloat16),
            (((1,), (1,)), ((), ())),
            preferred_element_type=jnp.float32)
        se[:, k:k + 1] += jnp.sum(jnp.exp(d * inv_t - inv_t), axis=1,
                                  keepdims=True)

    @pl.when(j == _N_STEPS - 1)
    def _fini():
        lse = jnp.log(se[...]) + inv_t
        lse_ref[...] = jnp.sum(lse, axis=0, keepdims=True) / BATCH


def _gold_kernel(x_rgb_ref, x_nir_ref, x_tir_ref,
                 g_rgb_ref, g_nir_ref, g_tir_ref, lse_ref,
                 o_rgb_ref, o_nir_ref, o_tir_ref):
    inv_t = 1.0 / TEMP
    for k, (x_ref, g_ref, o_ref) in enumerate((
            (x_rgb_ref, g_rgb_ref, o_rgb_ref),
            (x_nir_ref, g_nir_ref, o_nir_ref),
            (x_tir_ref, g_tir_ref, o_tir_ref))):
        x = x_ref[...]
        n = jnp.sqrt(jnp.sum(x * x, axis=1, keepdims=True))
        xn = x / jnp.maximum(n, 1e-12)
        gold = jnp.sum(xn * g_ref[...], axis=1, keepdims=True) * inv_t
        mean_gold = jnp.sum(gold, axis=0, keepdims=True) / BATCH
        o_ref[...] = lse_ref[:, k:k + 1] - mean_gold


@jax.jit
def kernel(inputs_rgb, inputs_nir, inputs_tir, targets,
           features_rgb, features_nir, features_tir):
    tgt = targets.astype(jnp.int32)
    g_rgb, g_nir, g_tir = _sc_gather(tgt, features_rgb,
                                     features_nir, features_tir)

    batch_spec = pl.BlockSpec((BATCH, NUM_FEATURES), lambda j: (0, 0))
    bank_spec = pl.BlockSpec((BLK, NUM_FEATURES), lambda j: (j, 0))

    lse = pl.pallas_call(
        _lse_kernel,
        grid=(_N_STEPS,),
        in_specs=[batch_spec, batch_spec, batch_spec,
                  bank_spec, bank_spec, bank_spec],
        out_specs=pl.BlockSpec((1, 3), lambda j: (0, 0)),
        out_shape=jax.ShapeDtypeStruct((1, 3), jnp.float32),
        scratch_shapes=[
            pltpu.VMEM((BATCH, NUM_FEATURES), jnp.float32),
            pltpu.VMEM((BATCH, NUM_FEATURES), jnp.float32),
            pltpu.VMEM((BATCH, NUM_FEATURES), jnp.float32),
            pltpu.VMEM((BATCH, 3), jnp.float32),
        ],
        compiler_params=pltpu.CompilerParams(
            dimension_semantics=("arbitrary",)),
    )(inputs_rgb, inputs_nir, inputs_tir,
      features_rgb, features_nir, features_tir)

    full = pl.BlockSpec((BATCH, NUM_FEATURES), lambda: (0, 0))
    scalar_spec = pl.BlockSpec((1, 1), lambda: (0, 0))
    o_rgb, o_nir, o_tir = pl.pallas_call(
        _gold_kernel,
        in_specs=[full, full, full, full, full, full,
                  pl.BlockSpec((1, 3), lambda: (0, 0))],
        out_specs=[scalar_spec, scalar_spec, scalar_spec],
        out_shape=[jax.ShapeDtypeStruct((1, 1), jnp.float32)] * 3,
    )(inputs_rgb, inputs_nir, inputs_tir, g_rgb, g_nir, g_tir, lse)

    return (o_rgb[0, 0], o_nir[0, 0], o_tir[0, 0])
